# baseline (speedup 1.0000x reference)
import functools

import jax
import jax.numpy as jnp
from jax import lax
from jax.experimental import pallas as pl
from jax.experimental.pallas import tpu as pltpu
from jax.experimental.pallas import tpu_sc as plsc

_N = 4 * 1 * 224 * 224
_NS = 16
_CHUNK = _N // _NS
_LANES = 16
_VREGS = _CHUNK // _LANES
_THRESH = 0.2
_INV_N = 1.0 / _N


@jax.jit
def _sc_masked_mean(x_flat):
    mesh = plsc.VectorSubcoreMesh(
        core_axis_name="c", subcore_axis_name="s", num_cores=1
    )

    @functools.partial(
        pl.kernel,
        mesh=mesh,
        out_type=jax.ShapeDtypeStruct((_LANES,), jnp.float32),
        scratch_types=[
            pltpu.VMEM((_CHUNK,), jnp.float32),
            pltpu.VMEM((1, _LANES), jnp.float32),
            pltpu.VMEM((1,), jnp.int32),
            pltpu.VMEM_SHARED((1, _LANES), jnp.float32),
        ],
    )
    def body(x_hbm, out_hbm, x_v, part_v, idx_v, shared):
        sid = lax.axis_index("s")

        @pl.when(sid == 0)
        def _():
            part_v[...] = jnp.zeros((1, _LANES), jnp.float32)
            pltpu.sync_copy(part_v, shared)

        pltpu.sync_copy(x_hbm.at[pl.ds(sid * _CHUNK, _CHUNK)], x_v)

        def step(i, acc):
            v = jnp.abs(x_v[pl.ds(i * _LANES, _LANES)])
            return acc + jnp.where(v < _THRESH, v, 0.0)

        acc = lax.fori_loop(
            0, _VREGS, step, jnp.zeros((_LANES,), jnp.float32)
        )
        plsc.subcore_barrier()
        idx_v[...] = jnp.zeros((1,), jnp.int32)
        part_v[0] = acc
        pltpu.sync_copy(part_v, shared.at[idx_v], add=True)
        plsc.subcore_barrier()

        @pl.when(sid == 0)
        def _():
            pltpu.sync_copy(shared, part_v)
            total = part_v[0]
            s = jnp.float32(0.0)
            for j in range(_LANES):
                s = s + total[j]
            part_v[0] = jnp.full((_LANES,), s * _INV_N, jnp.float32)
            pltpu.sync_copy(part_v.at[0], out_hbm)

    return body(x_flat)


def kernel(up, left, right):
    del left, right
    out = _sc_masked_mean(up.reshape(-1))
    return out[0]


# speedup vs baseline: 1.1393x; 1.1393x over previous
import functools

import jax
import jax.numpy as jnp
from jax import lax
from jax.experimental import pallas as pl
from jax.experimental.pallas import tpu as pltpu
from jax.experimental.pallas import tpu_sc as plsc

_N = 4 * 1 * 224 * 224
_NS = 16
_CHUNK = _N // _NS
_LANES = 16
_VREGS = _CHUNK // _LANES
_THRESH = 0.2
_INV_N = 1.0 / _N
_UNROLL = 8


@jax.jit
def _sc_masked_mean(x_flat):
    mesh = plsc.VectorSubcoreMesh(
        core_axis_name="c", subcore_axis_name="s", num_cores=1
    )

    @functools.partial(
        pl.kernel,
        mesh=mesh,
        out_type=jax.ShapeDtypeStruct((_LANES,), jnp.float32),
        scratch_types=[
            pltpu.VMEM((_CHUNK,), jnp.float32),
            pltpu.VMEM((1, _LANES), jnp.float32),
            pltpu.VMEM((1,), jnp.int32),
            pltpu.VMEM_SHARED((1, _LANES), jnp.float32),
        ],
    )
    def body(x_hbm, out_hbm, x_v, part_v, idx_v, shared):
        sid = lax.axis_index("s")

        @pl.when(sid == 0)
        def _():
            part_v[...] = jnp.zeros((1, _LANES), jnp.float32)
            pltpu.sync_copy(part_v, shared)

        pltpu.sync_copy(x_hbm.at[pl.ds(sid * _CHUNK, _CHUNK)], x_v)

        def step(i, accs):
            base = i * (_LANES * _UNROLL)
            out = []
            for k in range(_UNROLL):
                v = jnp.abs(x_v[pl.ds(base + k * _LANES, _LANES)])
                out.append(accs[k] + jnp.where(v < _THRESH, v, 0.0))
            return tuple(out)

        zero = jnp.zeros((_LANES,), jnp.float32)
        accs = lax.fori_loop(0, _VREGS // _UNROLL, step, (zero,) * _UNROLL)
        acc = zero
        for k in range(_UNROLL):
            acc = acc + accs[k]
        plsc.subcore_barrier()
        idx_v[...] = jnp.zeros((1,), jnp.int32)
        part_v[0] = acc
        pltpu.sync_copy(part_v, shared.at[idx_v], add=True)
        plsc.subcore_barrier()

        @pl.when(sid == 0)
        def _():
            pltpu.sync_copy(shared, part_v)
            total = part_v[0]
            s = jnp.float32(0.0)
            for j in range(_LANES):
                s = s + total[j]
            part_v[0] = jnp.full((_LANES,), s * _INV_N, jnp.float32)
            pltpu.sync_copy(part_v.at[0], out_hbm)

    return body(x_flat)


def kernel(up, left, right):
    del left, right
    out = _sc_masked_mean(up.reshape(-1))
    return out[0]
